# baseline (speedup 1.0000x reference)
import functools

import jax
import jax.numpy as jnp
from jax import lax
from jax.experimental import pallas as pl
from jax.experimental.pallas import tpu as pltpu
from jax.experimental.pallas import tpu_sc as plsc


def _emb_call(B, D, NC, NS):
    NW = NC * NS
    b_per_w = B // NW
    L = 16
    G = b_per_w // L
    LAG = 8
    mesh = plsc.VectorSubcoreMesh(core_axis_name="c", subcore_axis_name="s")

    @functools.partial(
        pl.kernel,
        mesh=mesh,
        out_type=jax.ShapeDtypeStruct((B, D), jnp.float32),
        compiler_params=pltpu.CompilerParams(use_tc_tiling_on_sc=False),
        scratch_types=[
            pltpu.VMEM((b_per_w,), jnp.int32),
            pltpu.SemaphoreType.DMA,
        ],
    )
    def emb(idx_hbm, table_hbm, out_hbm, idx_v, sem):
        wid = lax.axis_index("s") * NC + lax.axis_index("c")
        base = wid * b_per_w
        pltpu.sync_copy(idx_hbm.at[pl.ds(base, b_per_w)], idx_v)
        lanes = lax.broadcasted_iota(jnp.int32, (L,), 0)

        def group(g, carry):
            vec = idx_v[pl.ds(g * L, L)]
            for l in range(L):
                val = vec[l]
                pltpu.async_copy(
                    table_hbm.at[pl.ds(val, 1)],
                    out_hbm.at[pl.ds(base + g * L + l, 1)],
                    sem,
                )
            @pl.when(g >= LAG)
            def _():
                pltpu.make_async_copy(
                    table_hbm.at[pl.ds(0, L)], out_hbm.at[pl.ds(0, L)], sem
                ).wait()

            return carry

        lax.fori_loop(0, G, group, 0)
        for _ in range(LAG):
            pltpu.make_async_copy(
                table_hbm.at[pl.ds(0, L)], out_hbm.at[pl.ds(0, L)], sem
            ).wait()

    return emb


def kernel(input_sentences, embedding_weight):
    S0, S1 = input_sentences.shape
    V, D = embedding_weight.shape
    B = S0 * S1
    info = plsc.get_sparse_core_info()
    NC, NS = info.num_cores, info.num_subcores
    idx = input_sentences.reshape(B).astype(jnp.int32)
    out = _emb_call(B, D, NC, NS)(idx, embedding_weight)
    return out.reshape(S0, S1, D)


# speedup vs baseline: 3.2170x; 3.2170x over previous
import functools

import jax
import jax.numpy as jnp
from jax import lax
from jax.experimental import pallas as pl
from jax.experimental.pallas import tpu as pltpu
from jax.experimental.pallas import tpu_sc as plsc

_L = 16
_SPAN = 40
_C = 16


def _emb_call(B, D, VW, NC, NS):
    NW = NC * NS
    b_per_w = B // NW
    nch = b_per_w // _C
    npair = nch // 2
    rem = D % _L
    phase = D % _L
    mesh = plsc.VectorSubcoreMesh(core_axis_name="c", subcore_axis_name="s")
    g_bytes_rows = _C * _SPAN

    @functools.partial(
        pl.kernel,
        mesh=mesh,
        out_type=jax.ShapeDtypeStruct((B * D // _L, _L), jnp.float32),
        compiler_params=pltpu.CompilerParams(
            use_tc_tiling_on_sc=False, needs_layout_passes=False),
        scratch_types=[
            pltpu.VMEM((b_per_w * _SPAN,), jnp.int32),
            pltpu.VMEM((b_per_w,), jnp.int32),
            pltpu.VMEM((g_bytes_rows, _L), jnp.float32),
            pltpu.VMEM((g_bytes_rows, _L), jnp.float32),
            pltpu.VMEM(((_C * D + _L) // _L, _L), jnp.float32),
            pltpu.VMEM(((_C * D + _L) // _L, _L), jnp.float32),
            pltpu.SemaphoreType.DMA,
            pltpu.SemaphoreType.DMA,
            pltpu.SemaphoreType.DMA,
            pltpu.SemaphoreType.DMA,
        ],
    )
    def emb(idxe_hbm, idx_hbm, view_hbm, out_hbm,
            idxe_v, idx_v, st0, st1, dst0, dst1,
            gsem0, gsem1, ssem0, ssem1):
        wid = lax.axis_index("s") * NC + lax.axis_index("c")
        base = wid * b_per_w
        lanes = lax.broadcasted_iota(jnp.int32, (_L,), 0)
        pltpu.sync_copy(idxe_hbm.at[pl.ds(base * _SPAN, b_per_w * _SPAN)], idxe_v)
        pltpu.sync_copy(idx_hbm.at[pl.ds(base, b_per_w)], idx_v)

        def start_gather(chunk, st, gsem):
            pltpu.async_copy(
                view_hbm.at[idxe_v.at[pl.ds(chunk * (_C * _SPAN), _C * _SPAN)]],
                st, gsem)

        def wait_gather(st, gsem):
            pltpu.make_async_copy(
                view_hbm.at[pl.ds(0, g_bytes_rows)], st, gsem).wait()

        def start_store(chunk, dst, ssem):
            pltpu.async_copy(
                dst.at[pl.ds(0, _C * D // _L)],
                out_hbm.at[pl.ds((base + chunk * _C) * D // _L, _C * D // _L)],
                ssem)

        def wait_store(dst, ssem):
            pltpu.make_async_copy(
                out_hbm.at[pl.ds(0, _C * D // _L)],
                dst.at[pl.ds(0, _C * D // _L)], ssem).wait()

        def realign(chunk, st, dst):
            vec = idx_v[pl.ds(chunk * _C, _C)]
            t = []
            for r in range(_C):
                i = vec[r]
                o = (i * jnp.int32(phase)) & (_L - 1)
                t.append(o + jnp.int32((_SPAN * _L - D) * r))
            for q in range(_C * D // _L):
                w0 = _L * q
                r0 = w0 // D
                r1 = (w0 + _L - 1) // D
                if r1 == r0:
                    b = t[r0] + jnp.int32(w0)
                else:
                    lsplit = D * (r0 + 1) - w0
                    b = jnp.where(lanes >= lsplit,
                                  t[r1] + jnp.int32(w0), t[r0] + jnp.int32(w0))
                fv = lanes + b
                vals = plsc.load_gather(st, [fv >> 4, fv & 15])
                dst[q] = vals

        start_gather(0, st0, gsem0)

        def pair(t, carry):
            ca = 2 * t
            cb = 2 * t + 1
            start_gather(cb, st1, gsem1)
            wait_gather(st0, gsem0)

            @pl.when(t > 0)
            def _():
                wait_store(dst0, ssem0)

            realign(ca, st0, dst0)
            start_store(ca, dst0, ssem0)

            @pl.when(t < npair - 1)
            def _():
                start_gather(ca + 2, st0, gsem0)

            wait_gather(st1, gsem1)

            @pl.when(t > 0)
            def _():
                wait_store(dst1, ssem1)

            realign(cb, st1, dst1)
            start_store(cb, dst1, ssem1)
            return carry

        lax.fori_loop(0, npair, pair, 0)
        wait_store(dst0, ssem0)
        wait_store(dst1, ssem1)

    return emb


def kernel(input_sentences, embedding_weight):
    S0, S1 = input_sentences.shape
    V, D = embedding_weight.shape
    B = S0 * S1
    VW = V * D // _L
    info = plsc.get_sparse_core_info()
    NC, NS = info.num_cores, info.num_subcores
    idx = input_sentences.reshape(B).astype(jnp.int32)
    base_words = (idx * jnp.int32(D)) // _L
    idx_exp = jnp.minimum(
        base_words[:, None] + jnp.arange(_SPAN, dtype=jnp.int32)[None, :],
        jnp.int32(VW - 1)).reshape(-1)
    view = embedding_weight.reshape(VW, _L)
    out = _emb_call(B, D, VW, NC, NS)(idx_exp, idx, view)
    return out.reshape(S0, S1, D)
